# baseline (speedup 1.0000x reference)
import functools

import jax
import jax.numpy as jnp
from jax import lax
from jax.experimental import pallas as pl
from jax.experimental.pallas import tpu as pltpu
from jax.experimental.pallas import tpu_sc as plsc

VOCAB = 1_000_000
DIM = 64
BATCH = 16384
HIST = 50
NOUT = 2

MM_BLK = 25000
MM_GRID = VOCAB // MM_BLK

NW = 32
ROWS_PER_W = BATCH // NW
CB = 32
CHUNKS_PER_W = ROWS_PER_W // CB
NCHUNKS = BATCH // CB
EPR = HIST * NOUT
RSTRIDE = 112
LANES = 16
KR = RSTRIDE // LANES
GROUPS = CB * NOUT // LANES
RPG = LANES // NOUT


def _tc_project_table(table, Ws):
    def mm(t_ref, w_ref, o_ref):
        o_ref[...] = jnp.dot(t_ref[...], w_ref[...],
                             preferred_element_type=jnp.float32)

    return pl.pallas_call(
        mm,
        grid=(MM_GRID,),
        in_specs=[
            pl.BlockSpec((MM_BLK, DIM), lambda i: (i, 0)),
            pl.BlockSpec((DIM, NOUT), lambda i: (0, 0)),
        ],
        out_specs=pl.BlockSpec((MM_BLK, NOUT), lambda i: (i, 0)),
        out_shape=jax.ShapeDtypeStruct((VOCAB, NOUT), jnp.float32),
    )(table, Ws)


def _sc_lookup_pool(xi, p1d, b16):
    mesh = plsc.VectorSubcoreMesh(core_axis_name="c", subcore_axis_name="s")
    nc = mesh.num_cores

    @functools.partial(
        pl.kernel,
        out_type=jax.ShapeDtypeStruct((NCHUNKS, CB * NOUT), jnp.float32),
        mesh=mesh,
        scratch_types=[
            pltpu.VMEM((CB, EPR), jnp.int32),
            pltpu.VMEM((CB * RSTRIDE,), jnp.float32),
            pltpu.VMEM((CB * NOUT,), jnp.float32),
            pltpu.VMEM((LANES,), jnp.float32),
            pltpu.SemaphoreType.DMA,
        ],
        compiler_params=pltpu.CompilerParams(use_tc_tiling_on_sc=False),
    )
    def k(x_hbm, p_hbm, b16_hbm, out_hbm, idx_v, rows_v, ost_v, bias_v, sem):
        wid = lax.axis_index("s") * nc + lax.axis_index("c")
        pltpu.sync_copy(b16_hbm, bias_v)
        bias = bias_v[...]
        iota = lax.iota(jnp.int32, LANES)
        zeros = jnp.zeros((LANES,), jnp.float32)
        for r in range(CB):
            rows_v[pl.ds(r * RSTRIDE + 6 * LANES, LANES)] = zeros
        gdn = lax.GatherDimensionNumbers(
            offset_dims=(), collapsed_slice_dims=(0,), start_index_map=(0,))

        def lane_take(v, idx):
            return lax.gather(
                v, idx.reshape(LANES, 1), dimension_numbers=gdn,
                slice_sizes=(1,),
                mode=lax.GatherScatterMode.PROMISE_IN_BOUNDS)

        rot = [(iota + (1 << (s + 1))) & (LANES - 1) for s in range(3)]
        rmask = [(iota >> 1) == rr for rr in range(RPG)]

        def chunk_body(g, carry):
            chunk = wid * CHUNKS_PER_W + g
            pltpu.sync_copy(x_hbm.at[chunk], idx_v)
            copies = [
                pltpu.async_copy(
                    p_hbm.at[idx_v.at[r]],
                    rows_v.at[pl.ds(r * RSTRIDE, EPR)],
                    sem,
                )
                for r in range(CB)
            ]
            for cpy in copies:
                cpy.wait()

            for grp in range(GROUPS):
                ost = bias
                for rr in range(RPG):
                    base = (grp * RPG + rr) * RSTRIDE
                    acc = rows_v[pl.ds(base, LANES)]
                    for k16 in range(1, KR):
                        acc = acc + rows_v[pl.ds(base + k16 * LANES, LANES)]
                    for s in range(3):
                        acc = acc + lane_take(acc, rot[s])
                    ost = ost + jnp.where(rmask[rr], acc, 0.0)
                ost_v[pl.ds(grp * LANES, LANES)] = ost

            pltpu.sync_copy(ost_v, out_hbm.at[chunk])
            return carry

        lax.fori_loop(0, CHUNKS_PER_W, chunk_body, 0)

    return k(xi, p1d, b16)


def kernel(x, table, W, b):
    xe = x.astype(jnp.int32) * NOUT
    xi = jnp.stack([xe, xe + 1], axis=-1).reshape(NCHUNKS, CB, EPR)
    p = _tc_project_table(table, W * (1.0 / HIST))
    b16 = jnp.tile(b.astype(jnp.float32), RPG)
    out = _sc_lookup_pool(xi, p.reshape(VOCAB * NOUT), b16)
    return out.reshape(BATCH, NOUT)


# speedup vs baseline: 1.0421x; 1.0421x over previous
import functools

import jax
import jax.numpy as jnp
from jax import lax
from jax.experimental import pallas as pl
from jax.experimental.pallas import tpu as pltpu
from jax.experimental.pallas import tpu_sc as plsc

VOCAB = 1_000_000
DIM = 64
BATCH = 16384
HIST = 50
NOUT = 2

T128_ROWS = VOCAB // 2
MM_BLK = 16000
MM_GRID = -(-T128_ROWS // MM_BLK)

NW = 32
ROWS_PER_W = BATCH // NW
CB = 32
CHUNKS_PER_W = ROWS_PER_W // CB
NCHUNKS = BATCH // CB
EPR = HIST * NOUT
RSTRIDE = 112
LANES = 16
KR = RSTRIDE // LANES
GROUPS = CB * NOUT // LANES
RPG = LANES // NOUT


def _tc_project_table(table128, W4):
    def mm(t_ref, w_ref, o_ref):
        o_ref[...] = jnp.dot(t_ref[...], w_ref[...],
                             preferred_element_type=jnp.float32)

    return pl.pallas_call(
        mm,
        grid=(MM_GRID,),
        in_specs=[
            pl.BlockSpec((MM_BLK, 2 * DIM), lambda i: (i, 0)),
            pl.BlockSpec((2 * DIM, 2 * NOUT), lambda i: (0, 0)),
        ],
        out_specs=pl.BlockSpec((MM_BLK, 2 * NOUT), lambda i: (i, 0)),
        out_shape=jax.ShapeDtypeStruct((T128_ROWS, 2 * NOUT), jnp.float32),
    )(table128, W4)


def _sc_lookup_pool(xi, p1d, b16):
    mesh = plsc.VectorSubcoreMesh(core_axis_name="c", subcore_axis_name="s")
    nc = mesh.num_cores

    @functools.partial(
        pl.kernel,
        out_type=jax.ShapeDtypeStruct((NCHUNKS, CB * NOUT), jnp.float32),
        mesh=mesh,
        scratch_types=[
            pltpu.VMEM((CB, EPR), jnp.int32),
            pltpu.VMEM((CB * RSTRIDE,), jnp.float32),
            pltpu.VMEM((CB * NOUT,), jnp.float32),
            pltpu.VMEM((LANES,), jnp.float32),
            pltpu.SemaphoreType.DMA,
        ],
        compiler_params=pltpu.CompilerParams(use_tc_tiling_on_sc=False),
    )
    def k(x_hbm, p_hbm, b16_hbm, out_hbm, idx_v, rows_v, ost_v, bias_v, sem):
        wid = lax.axis_index("s") * nc + lax.axis_index("c")
        pltpu.sync_copy(b16_hbm, bias_v)
        bias = bias_v[...]
        iota = lax.iota(jnp.int32, LANES)
        zeros = jnp.zeros((LANES,), jnp.float32)
        for r in range(CB):
            rows_v[pl.ds(r * RSTRIDE + 6 * LANES, LANES)] = zeros
        gdn = lax.GatherDimensionNumbers(
            offset_dims=(), collapsed_slice_dims=(0,), start_index_map=(0,))

        def lane_take(v, idx):
            return lax.gather(
                v, idx.reshape(LANES, 1), dimension_numbers=gdn,
                slice_sizes=(1,),
                mode=lax.GatherScatterMode.PROMISE_IN_BOUNDS)

        rot = [(iota + (1 << (s + 1))) & (LANES - 1) for s in range(3)]
        rmask = [(iota >> 1) == rr for rr in range(RPG)]

        def chunk_body(g, carry):
            chunk = wid * CHUNKS_PER_W + g
            pltpu.sync_copy(x_hbm.at[chunk], idx_v)
            copies = [
                pltpu.async_copy(
                    p_hbm.at[idx_v.at[r]],
                    rows_v.at[pl.ds(r * RSTRIDE, EPR)],
                    sem,
                )
                for r in range(CB)
            ]
            for cpy in copies:
                cpy.wait()

            for grp in range(GROUPS):
                ost = bias
                for rr in range(RPG):
                    base = (grp * RPG + rr) * RSTRIDE
                    acc = rows_v[pl.ds(base, LANES)]
                    for k16 in range(1, KR):
                        acc = acc + rows_v[pl.ds(base + k16 * LANES, LANES)]
                    for s in range(3):
                        acc = acc + lane_take(acc, rot[s])
                    ost = ost + jnp.where(rmask[rr], acc, 0.0)
                ost_v[pl.ds(grp * LANES, LANES)] = ost

            pltpu.sync_copy(ost_v, out_hbm.at[chunk])
            return carry

        lax.fori_loop(0, CHUNKS_PER_W, chunk_body, 0)

    return k(xi, p1d, b16)


def kernel(x, table, W, b):
    xe = x.astype(jnp.int32) * NOUT
    xi = jnp.stack([xe, xe + 1], axis=-1).reshape(NCHUNKS, CB, EPR)
    Ws = W.astype(jnp.float32) * (1.0 / HIST)
    W4 = (jnp.zeros((2 * DIM, 2 * NOUT), jnp.float32)
          .at[:DIM, :NOUT].set(Ws).at[DIM:, NOUT:].set(Ws))
    p4 = _tc_project_table(table.reshape(T128_ROWS, 2 * DIM), W4)
    b16 = jnp.tile(b.astype(jnp.float32), RPG)
    out = _sc_lookup_pool(xi, p4.reshape(VOCAB * NOUT), b16)
    return out.reshape(BATCH, NOUT)


# speedup vs baseline: 1.4350x; 1.3770x over previous
import functools

import jax
import jax.numpy as jnp
from jax import lax
from jax.experimental import pallas as pl
from jax.experimental.pallas import tpu as pltpu
from jax.experimental.pallas import tpu_sc as plsc

VOCAB = 1_000_000
DIM = 64
BATCH = 16384
HIST = 50

NW = 32
ROWS_PER_W = BATCH // NW
CB = 16
CHUNKS_PER_W = ROWS_PER_W // CB
NCHUNKS = BATCH // CB
IDX_PER_CHUNK = CB * HIST
GW = 80
NG = IDX_PER_CHUNK // GW
LANES = 16
KREG = DIM // LANES


def _sc_gather_pool(x3, table):
    mesh = plsc.VectorSubcoreMesh(core_axis_name="c", subcore_axis_name="s")
    nc = mesh.num_cores

    @functools.partial(
        pl.kernel,
        out_type=jax.ShapeDtypeStruct((NCHUNKS, CB, DIM), jnp.float32),
        mesh=mesh,
        scratch_types=[
            pltpu.VMEM((NG, GW), jnp.int32),
            pltpu.VMEM((IDX_PER_CHUNK, DIM), jnp.float32),
            pltpu.VMEM((CB, DIM), jnp.float32),
            pltpu.SemaphoreType.DMA,
        ],
        compiler_params=pltpu.CompilerParams(use_tc_tiling_on_sc=False),
    )
    def k(x_hbm, table_hbm, out_hbm, idx_v, rows_v, pooled_v, sem):
        wid = lax.axis_index("s") * nc + lax.axis_index("c")

        def chunk_body(g, carry):
            chunk = wid * CHUNKS_PER_W + g
            pltpu.sync_copy(x_hbm.at[chunk], idx_v)
            copies = [
                pltpu.async_copy(
                    table_hbm.at[idx_v.at[j]],
                    rows_v.at[pl.ds(j * GW, GW)],
                    sem,
                )
                for j in range(NG)
            ]
            for cpy in copies:
                cpy.wait()

            def row_body(bi, rcarry):
                base = bi * HIST
                accs = [rows_v[base, pl.ds(k16 * LANES, LANES)]
                        for k16 in range(KREG)]
                for l in range(1, HIST):
                    for k16 in range(KREG):
                        accs[k16] = accs[k16] + rows_v[
                            base + l, pl.ds(k16 * LANES, LANES)]
                for k16 in range(KREG):
                    pooled_v[bi, pl.ds(k16 * LANES, LANES)] = accs[k16]
                return rcarry

            lax.fori_loop(0, CB, row_body, 0)
            pltpu.sync_copy(pooled_v, out_hbm.at[chunk])
            return carry

        lax.fori_loop(0, CHUNKS_PER_W, chunk_body, 0)

    return k(x3, table)


def _tc_project(pooled, W, bvec):
    def mm(p_ref, w_ref, b_ref, o_ref):
        o_ref[...] = (
            jnp.dot(p_ref[...], w_ref[...],
                    preferred_element_type=jnp.float32) * (1.0 / HIST)
            + b_ref[...]
        )

    return pl.pallas_call(
        mm,
        out_shape=jax.ShapeDtypeStruct((BATCH, 2), jnp.float32),
    )(pooled, W, bvec)


def kernel(x, table, W, b):
    x3 = x.astype(jnp.int32).reshape(NCHUNKS, NG, GW)
    pooled = _sc_gather_pool(x3, table).reshape(BATCH, DIM)
    return _tc_project(pooled, W, b.reshape(1, 2))


# speedup vs baseline: 1.5831x; 1.1032x over previous
import functools

import jax
import jax.numpy as jnp
from jax import lax
from jax.experimental import pallas as pl
from jax.experimental.pallas import tpu as pltpu
from jax.experimental.pallas import tpu_sc as plsc

VOCAB = 1_000_000
DIM = 64
BATCH = 16384
HIST = 50

NW = 32
ROWS_PER_W = BATCH // NW
CB = 16
CHUNKS_PER_W = ROWS_PER_W // CB
NCHUNKS = BATCH // CB
IDX_PER_CHUNK = CB * HIST
GW = 80
NG = IDX_PER_CHUNK // GW
LANES = 16
KREG = DIM // LANES


def _sc_gather_pool(x3, table):
    mesh = plsc.VectorSubcoreMesh(core_axis_name="c", subcore_axis_name="s")
    nc = mesh.num_cores

    @functools.partial(
        pl.kernel,
        out_type=jax.ShapeDtypeStruct((NCHUNKS, CB, DIM), jnp.float32),
        mesh=mesh,
        scratch_types=[
            pltpu.VMEM((NG, GW), jnp.int32),
            pltpu.VMEM((NG, GW), jnp.int32),
            pltpu.VMEM((IDX_PER_CHUNK, DIM), jnp.float32),
            pltpu.VMEM((IDX_PER_CHUNK, DIM), jnp.float32),
            pltpu.VMEM((CB, DIM), jnp.float32),
            pltpu.SemaphoreType.DMA,
            pltpu.SemaphoreType.DMA,
        ],
        compiler_params=pltpu.CompilerParams(use_tc_tiling_on_sc=False),
    )
    def k(x_hbm, table_hbm, out_hbm, idx_a, idx_b, rows_a, rows_b,
          pooled_v, sem_a, sem_b):
        wid = lax.axis_index("s") * nc + lax.axis_index("c")

        def fire(g, idx_v, rows_v, sem):
            chunk = wid * CHUNKS_PER_W + g
            pltpu.sync_copy(x_hbm.at[chunk], idx_v)
            for j in range(NG):
                pltpu.async_copy(
                    table_hbm.at[idx_v.at[j]],
                    rows_v.at[pl.ds(j * GW, GW)],
                    sem,
                )

        def drain(idx_v, rows_v, sem):
            for j in range(NG):
                pltpu.make_async_copy(
                    table_hbm.at[idx_v.at[j]],
                    rows_v.at[pl.ds(j * GW, GW)],
                    sem,
                ).wait()

        def consume(g, rows_v):
            chunk = wid * CHUNKS_PER_W + g

            def row_body(bi, rcarry):
                base = bi * HIST
                accs = [rows_v[base, pl.ds(k16 * LANES, LANES)]
                        for k16 in range(KREG)]
                for l in range(1, HIST):
                    for k16 in range(KREG):
                        accs[k16] = accs[k16] + rows_v[
                            base + l, pl.ds(k16 * LANES, LANES)]
                for k16 in range(KREG):
                    pooled_v[bi, pl.ds(k16 * LANES, LANES)] = accs[k16]
                return rcarry

            lax.fori_loop(0, CB, row_body, 0)
            pltpu.sync_copy(pooled_v, out_hbm.at[chunk])

        fire(0, idx_a, rows_a, sem_a)

        def pair_body(i, carry):
            fire(2 * i + 1, idx_b, rows_b, sem_b)
            drain(idx_a, rows_a, sem_a)
            consume(2 * i, rows_a)

            @pl.when(i < CHUNKS_PER_W // 2 - 1)
            def _():
                fire(2 * i + 2, idx_a, rows_a, sem_a)

            drain(idx_b, rows_b, sem_b)
            consume(2 * i + 1, rows_b)
            return carry

        lax.fori_loop(0, CHUNKS_PER_W // 2, pair_body, 0)

    return k(x3, table)


def _tc_project(pooled, W, bvec):
    def mm(p_ref, w_ref, b_ref, o_ref):
        o_ref[...] = (
            jnp.dot(p_ref[...], w_ref[...],
                    preferred_element_type=jnp.float32) * (1.0 / HIST)
            + b_ref[...]
        )

    return pl.pallas_call(
        mm,
        out_shape=jax.ShapeDtypeStruct((BATCH, 2), jnp.float32),
    )(pooled, W, bvec)


def kernel(x, table, W, b):
    x3 = x.astype(jnp.int32).reshape(NCHUNKS, NG, GW)
    pooled = _sc_gather_pool(x3, table).reshape(BATCH, DIM)
    return _tc_project(pooled, W, b.reshape(1, 2))
